# baseline (device time: 18384 ns/iter reference)
import jax
import jax.numpy as jnp
from jax import lax
from jax.experimental import pallas as pl
from jax.experimental.pallas import tpu as pltpu

N_DEV = 16
NBUF = 3


def kernel(x, w_mat):
    k_full, m_per = x.shape
    n = w_mat.shape[1]
    assert k_full == N_DEV * m_per

    def body(x_hbm, w_hbm, out_hbm, send_ref, comm_ref, wbuf_ref, xv_ref,
             outv_ref, send_sems, recv_sems, wdma_sems, x_sem, out_sem):
        my_i = lax.axis_index("i")

        x_dma = pltpu.make_async_copy(x_hbm, xv_ref, x_sem)
        x_dma.start()

        def src_of(t):
            return lax.rem(my_i - t + N_DEV, N_DEV)

        def w_dma(t):
            return pltpu.make_async_copy(
                w_hbm.at[pl.ds(src_of(t) * m_per, m_per), :],
                wbuf_ref.at[t],
                wdma_sems.at[t],
            )

        for t in range(N_DEV):
            w_dma(t).start()

        barrier_sem = pltpu.get_barrier_semaphore()
        for k in range(1, N_DEV):
            dst = lax.rem(my_i + k, N_DEV)
            pl.semaphore_signal(
                barrier_sem, inc=1,
                device_id=(dst,), device_id_type=pl.DeviceIdType.MESH,
            )

        x_dma.wait()
        for d in range(N_DEV):
            send_ref[d] = xv_ref[pl.ds(d * m_per, m_per), :].astype(jnp.bfloat16)
        comm_ref[my_i] = send_ref[my_i]

        w_dma(0).wait()
        acc = jnp.dot(
            comm_ref[my_i],
            wbuf_ref[0].astype(jnp.bfloat16),
            preferred_element_type=jnp.float32,
        )

        pl.semaphore_wait(barrier_sem, N_DEV - 1)

        for k in range(1, N_DEV):
            dst = lax.rem(my_i + k, N_DEV)
            pltpu.make_async_remote_copy(
                src_ref=send_ref.at[dst],
                dst_ref=comm_ref.at[my_i],
                send_sem=send_sems.at[dst],
                recv_sem=recv_sems.at[my_i],
                device_id=(dst,),
                device_id_type=pl.DeviceIdType.MESH,
            ).start()

        for t in range(1, N_DEV):
            j = src_of(t)
            pltpu.make_async_remote_copy(
                src_ref=send_ref.at[j],
                dst_ref=comm_ref.at[j],
                send_sem=send_sems.at[j],
                recv_sem=recv_sems.at[j],
                device_id=(my_i,),
                device_id_type=pl.DeviceIdType.MESH,
            ).wait_recv()
            w_dma(t).wait()
            acc = acc + jnp.dot(
                comm_ref[j],
                wbuf_ref[t].astype(jnp.bfloat16),
                preferred_element_type=jnp.float32,
            )

        outv_ref[...] = jnp.maximum(acc, 0.0)
        out_dma = pltpu.make_async_copy(outv_ref, out_hbm, out_sem)
        out_dma.start()

        for k in range(1, N_DEV):
            dst = lax.rem(my_i + k, N_DEV)
            pltpu.make_async_remote_copy(
                src_ref=send_ref.at[dst],
                dst_ref=comm_ref.at[my_i],
                send_sem=send_sems.at[dst],
                recv_sem=recv_sems.at[my_i],
                device_id=(dst,),
                device_id_type=pl.DeviceIdType.MESH,
            ).wait_send()
        out_dma.wait()

    return pl.pallas_call(
        body,
        out_shape=jax.ShapeDtypeStruct((m_per, n), jnp.float32),
        in_specs=[
            pl.BlockSpec(memory_space=pltpu.MemorySpace.HBM),
            pl.BlockSpec(memory_space=pltpu.MemorySpace.HBM),
        ],
        out_specs=pl.BlockSpec(memory_space=pltpu.MemorySpace.HBM),
        scratch_shapes=[
            pltpu.VMEM((N_DEV, m_per, m_per), jnp.bfloat16),
            pltpu.VMEM((N_DEV, m_per, m_per), jnp.bfloat16),
            pltpu.VMEM((N_DEV, m_per, n), jnp.float32),
            pltpu.VMEM((k_full, m_per), jnp.float32),
            pltpu.VMEM((m_per, n), jnp.float32),
            pltpu.SemaphoreType.DMA((N_DEV,)),
            pltpu.SemaphoreType.DMA((N_DEV,)),
            pltpu.SemaphoreType.DMA((N_DEV,)),
            pltpu.SemaphoreType.DMA,
            pltpu.SemaphoreType.DMA,
        ],
        compiler_params=pltpu.CompilerParams(collective_id=0),
    )(
        pltpu.with_memory_space_constraint(x, pltpu.MemorySpace.HBM),
        pltpu.with_memory_space_constraint(w_mat, pltpu.MemorySpace.HBM),
    )


# device time: 18217 ns/iter; 1.0092x vs baseline; 1.0092x over previous
import jax
import jax.numpy as jnp
from jax import lax
from jax.experimental import pallas as pl
from jax.experimental.pallas import tpu as pltpu

N_DEV = 16
NBUF = 3


def kernel(x, w_mat):
    k_full, m_per = x.shape
    n = w_mat.shape[1]
    assert k_full == N_DEV * m_per

    def body(x_hbm, w_hbm, out_ref, send_ref, comm_ref, wbuf_ref, xv_ref,
             send_sems, recv_sems, wdma_sems, x_sem):
        my_i = lax.axis_index("i")

        x_dma = pltpu.make_async_copy(x_hbm, xv_ref, x_sem)
        x_dma.start()

        def src_of(t):
            return lax.rem(my_i - t + N_DEV, N_DEV)

        def w_dma(t):
            return pltpu.make_async_copy(
                w_hbm.at[pl.ds(src_of(t) * m_per, m_per), :],
                wbuf_ref.at[t],
                wdma_sems.at[t],
            )

        for t in range(N_DEV):
            w_dma(t).start()

        barrier_sem = pltpu.get_barrier_semaphore()
        for k in range(1, N_DEV):
            dst = lax.rem(my_i + k, N_DEV)
            pl.semaphore_signal(
                barrier_sem, inc=1,
                device_id=(dst,), device_id_type=pl.DeviceIdType.MESH,
            )

        x_dma.wait()
        for d in range(N_DEV):
            send_ref[d] = xv_ref[pl.ds(d * m_per, m_per), :].astype(jnp.bfloat16)
        comm_ref[my_i] = send_ref[my_i]

        w_dma(0).wait()
        acc = jnp.dot(
            comm_ref[my_i],
            wbuf_ref[0].astype(jnp.bfloat16),
            preferred_element_type=jnp.float32,
        )

        pl.semaphore_wait(barrier_sem, N_DEV - 1)

        for k in range(1, N_DEV):
            dst = lax.rem(my_i + k, N_DEV)
            pltpu.make_async_remote_copy(
                src_ref=send_ref.at[dst],
                dst_ref=comm_ref.at[my_i],
                send_sem=send_sems.at[dst],
                recv_sem=recv_sems.at[my_i],
                device_id=(dst,),
                device_id_type=pl.DeviceIdType.MESH,
            ).start()

        for t in range(1, N_DEV):
            j = src_of(t)
            pltpu.make_async_remote_copy(
                src_ref=send_ref.at[j],
                dst_ref=comm_ref.at[j],
                send_sem=send_sems.at[j],
                recv_sem=recv_sems.at[j],
                device_id=(my_i,),
                device_id_type=pl.DeviceIdType.MESH,
            ).wait_recv()
            w_dma(t).wait()
            acc = acc + jnp.dot(
                comm_ref[j],
                wbuf_ref[t].astype(jnp.bfloat16),
                preferred_element_type=jnp.float32,
            )

        out_ref[...] = jnp.maximum(acc, 0.0)

        for k in range(1, N_DEV):
            dst = lax.rem(my_i + k, N_DEV)
            pltpu.make_async_remote_copy(
                src_ref=send_ref.at[dst],
                dst_ref=comm_ref.at[my_i],
                send_sem=send_sems.at[dst],
                recv_sem=recv_sems.at[my_i],
                device_id=(dst,),
                device_id_type=pl.DeviceIdType.MESH,
            ).wait_send()

    return pl.pallas_call(
        body,
        out_shape=jax.ShapeDtypeStruct((m_per, n), jnp.float32),
        in_specs=[
            pl.BlockSpec(memory_space=pltpu.MemorySpace.HBM),
            pl.BlockSpec(memory_space=pltpu.MemorySpace.HBM),
        ],
        out_specs=pl.BlockSpec(memory_space=pltpu.VMEM),
        scratch_shapes=[
            pltpu.VMEM((N_DEV, m_per, m_per), jnp.bfloat16),
            pltpu.VMEM((N_DEV, m_per, m_per), jnp.bfloat16),
            pltpu.VMEM((N_DEV, m_per, n), jnp.float32),
            pltpu.VMEM((k_full, m_per), jnp.float32),
            pltpu.SemaphoreType.DMA((N_DEV,)),
            pltpu.SemaphoreType.DMA((N_DEV,)),
            pltpu.SemaphoreType.DMA((N_DEV,)),
            pltpu.SemaphoreType.DMA,
        ],
        compiler_params=pltpu.CompilerParams(collective_id=0),
    )(
        pltpu.with_memory_space_constraint(x, pltpu.MemorySpace.HBM),
        pltpu.with_memory_space_constraint(w_mat, pltpu.MemorySpace.HBM),
    )


# device time: 14663 ns/iter; 1.2538x vs baseline; 1.2424x over previous
import jax
import jax.numpy as jnp
from jax import lax
from jax.experimental import pallas as pl
from jax.experimental.pallas import tpu as pltpu

N_DEV = 16
NG = 4
GSZ = N_DEV // NG


def kernel(x, w_mat):
    k_full, m_per = x.shape
    n = w_mat.shape[1]
    assert k_full == N_DEV * m_per
    kg = GSZ * m_per

    def body(x_ref, w_hbm, out_ref, send_ref, comm_ref, wbuf_ref, xg_ref,
             send_sems, recv_sems, wdma_sems):
        my_i = lax.axis_index("i")
        my_g = lax.div(my_i, GSZ)

        def grp_of(t):
            return lax.rem(my_g - t + NG, NG)

        wdmas = []
        for t in range(NG):
            dma = pltpu.make_async_copy(
                w_hbm.at[pl.ds(grp_of(t) * kg, kg), :],
                wbuf_ref.at[t],
                wdma_sems.at[t],
            )
            dma.start()
            wdmas.append(dma)

        barrier_sem = pltpu.get_barrier_semaphore()
        for k in range(1, N_DEV):
            dst = lax.rem(my_i + k, N_DEV)
            pl.semaphore_signal(
                barrier_sem, inc=1,
                device_id=(dst,), device_id_type=pl.DeviceIdType.MESH,
            )

        for d in range(N_DEV):
            send_ref[d] = x_ref[pl.ds(d * m_per, m_per), :].astype(jnp.bfloat16)
        comm_ref[my_i] = send_ref[my_i]

        pl.semaphore_wait(barrier_sem, N_DEV - 1)

        for k in range(1, N_DEV):
            dst = lax.rem(my_i + k, N_DEV)
            pltpu.make_async_remote_copy(
                src_ref=send_ref.at[dst],
                dst_ref=comm_ref.at[my_i],
                send_sem=send_sems.at[dst],
                recv_sem=recv_sems.at[my_i],
                device_id=(dst,),
                device_id_type=pl.DeviceIdType.MESH,
            ).start()

        acc = jnp.zeros((m_per, n), jnp.float32)
        for t in range(NG):
            g = grp_of(t)
            for u in range(GSZ):
                j = g * GSZ + u

                @pl.when(j != my_i)
                def _():
                    pltpu.make_async_remote_copy(
                        src_ref=send_ref.at[j],
                        dst_ref=comm_ref.at[j],
                        send_sem=send_sems.at[j],
                        recv_sem=recv_sems.at[j],
                        device_id=(my_i,),
                        device_id_type=pl.DeviceIdType.MESH,
                    ).wait_recv()

                xg_ref[t, :, pl.ds(u * m_per, m_per)] = comm_ref[j]

            wdmas[t].wait()
            acc = acc + jnp.dot(
                xg_ref[t],
                wbuf_ref[t].astype(jnp.bfloat16),
                preferred_element_type=jnp.float32,
            )

        out_ref[...] = jnp.maximum(acc, 0.0)

        for k in range(1, N_DEV):
            dst = lax.rem(my_i + k, N_DEV)
            pltpu.make_async_remote_copy(
                src_ref=send_ref.at[dst],
                dst_ref=comm_ref.at[my_i],
                send_sem=send_sems.at[dst],
                recv_sem=recv_sems.at[my_i],
                device_id=(dst,),
                device_id_type=pl.DeviceIdType.MESH,
            ).wait_send()

    return pl.pallas_call(
        body,
        out_shape=jax.ShapeDtypeStruct((m_per, n), jnp.float32),
        in_specs=[
            pl.BlockSpec(memory_space=pltpu.VMEM),
            pl.BlockSpec(memory_space=pltpu.MemorySpace.HBM),
        ],
        out_specs=pl.BlockSpec(memory_space=pltpu.VMEM),
        scratch_shapes=[
            pltpu.VMEM((N_DEV, m_per, m_per), jnp.bfloat16),
            pltpu.VMEM((N_DEV, m_per, m_per), jnp.bfloat16),
            pltpu.VMEM((NG, kg, n), jnp.float32),
            pltpu.VMEM((NG, m_per, kg), jnp.bfloat16),
            pltpu.SemaphoreType.DMA((N_DEV,)),
            pltpu.SemaphoreType.DMA((N_DEV,)),
            pltpu.SemaphoreType.DMA((NG,)),
        ],
        compiler_params=pltpu.CompilerParams(collective_id=0),
    )(x, pltpu.with_memory_space_constraint(w_mat, pltpu.MemorySpace.HBM))
